# baseline (device time: 69431 ns/iter reference)
import os

import jax
import jax.numpy as jnp
from jax import lax
from jax.experimental import pallas as pl
from jax.experimental.pallas import tpu as pltpu

M = 4096
N = 1024
H = M // 2
C = int(os.environ.get("CHUNKS", "8"))
R = H // C
VARIANT = os.environ.get("KERNEL_VARIANT", "interleave")


def kernel(x):
    assert x.shape == (M, N), x.shape

    def body(x_ref, out_ref, x_send_sems, x_recv_sems, y_send_sems, y_recv_sems):
        my_x = lax.axis_index("x")
        my_y = lax.axis_index("y")
        my_z = lax.axis_index("z")
        x_nbr = (1 - my_x, my_y, my_z)
        y_nbr = (my_x, 1 - my_y, my_z)

        barrier_sem = pltpu.get_barrier_semaphore()
        for nbr in (x_nbr, y_nbr):
            pl.semaphore_signal(
                barrier_sem, inc=1,
                device_id=nbr, device_id_type=pl.DeviceIdType.MESH,
            )
        pl.semaphore_wait(barrier_sem, 2)

        mine_send = my_x * M + my_y * H
        mine_keep = my_x * M + (1 - my_y) * H
        x_recv = (1 - my_x) * M + my_y * H

        def x_rdma(c):
            return pltpu.make_async_remote_copy(
                src_ref=out_ref.at[pl.ds(mine_send + c * R, R), :],
                dst_ref=out_ref.at[pl.ds(mine_send + c * R, R), :],
                send_sem=x_send_sems.at[c],
                recv_sem=x_recv_sems.at[c],
                device_id=x_nbr,
                device_id_type=pl.DeviceIdType.MESH,
            )

        x_rdmas = []
        if VARIANT == "upfront":
            out_ref[pl.ds(my_x * M, M), :] = x_ref[:, :].astype(jnp.bfloat16)
            for c in range(C):
                rdma = x_rdma(c)
                rdma.start()
                x_rdmas.append(rdma)
        else:
            for c in range(C):
                out_ref[pl.ds(mine_send + c * R, R), :] = (
                    x_ref[pl.ds(my_y * H + c * R, R), :].astype(jnp.bfloat16)
                )
                rdma = x_rdma(c)
                rdma.start()
                x_rdmas.append(rdma)
            out_ref[pl.ds(mine_keep, H), :] = (
                x_ref[pl.ds((1 - my_y) * H, H), :].astype(jnp.bfloat16)
            )

        y_rdmas = []
        for c in range(C):
            x_rdmas[c].wait_recv()
            rdma = pltpu.make_async_remote_copy(
                src_ref=out_ref.at[pl.ds(x_recv + c * R, R), :],
                dst_ref=out_ref.at[pl.ds(x_recv + c * R, R), :],
                send_sem=y_send_sems.at[c],
                recv_sem=y_recv_sems.at[c],
                device_id=y_nbr,
                device_id_type=pl.DeviceIdType.MESH,
            )
            rdma.start()
            y_rdmas.append(rdma)

        for c in range(C):
            y_rdmas[c].wait_recv()
        for c in range(C):
            x_rdmas[c].wait_send()
            y_rdmas[c].wait_send()

    return pl.pallas_call(
        body,
        out_shape=jax.ShapeDtypeStruct((2 * M, N), jnp.bfloat16),
        in_specs=[pl.BlockSpec(memory_space=pltpu.VMEM)],
        out_specs=pl.BlockSpec(memory_space=pltpu.VMEM),
        scratch_shapes=[
            pltpu.SemaphoreType.DMA((C,)),
            pltpu.SemaphoreType.DMA((C,)),
            pltpu.SemaphoreType.DMA((C,)),
            pltpu.SemaphoreType.DMA((C,)),
        ],
        compiler_params=pltpu.CompilerParams(collective_id=0),
    )(x)


# device time: 60606 ns/iter; 1.1456x vs baseline; 1.1456x over previous
import os

import jax
import jax.numpy as jnp
from jax import lax
from jax.experimental import pallas as pl
from jax.experimental.pallas import tpu as pltpu

M = 4096
N = 1024
H = M // 2
C = int(os.environ.get("CHUNKS", "8"))
R = H // C
VARIANT = os.environ.get("KERNEL_VARIANT", "interleave")


def kernel(x):
    assert x.shape == (M, N), x.shape

    def body(x_ref, out_ref, x_send_sems, x_recv_sems, y_send_sems, y_recv_sems):
        my_x = lax.axis_index("x")
        my_y = lax.axis_index("y")
        my_z = lax.axis_index("z")
        x_nbr = (1 - my_x, my_y, my_z)
        y_nbr = (my_x, 1 - my_y, my_z)

        barrier_sem = pltpu.get_barrier_semaphore()
        barrier_nbrs = (x_nbr,) if VARIANT == "halfx" else (x_nbr, y_nbr)
        for nbr in barrier_nbrs:
            pl.semaphore_signal(
                barrier_sem, inc=1,
                device_id=nbr, device_id_type=pl.DeviceIdType.MESH,
            )
        pl.semaphore_wait(barrier_sem, len(barrier_nbrs))

        mine_send = my_x * M + my_y * H
        mine_keep = my_x * M + (1 - my_y) * H
        x_recv = (1 - my_x) * M + my_y * H

        def x_rdma(c):
            return pltpu.make_async_remote_copy(
                src_ref=out_ref.at[pl.ds(mine_send + c * R, R), :],
                dst_ref=out_ref.at[pl.ds(mine_send + c * R, R), :],
                send_sem=x_send_sems.at[c],
                recv_sem=x_recv_sems.at[c],
                device_id=x_nbr,
                device_id_type=pl.DeviceIdType.MESH,
            )

        x_rdmas = []
        if VARIANT == "upfront":
            out_ref[pl.ds(my_x * M, M), :] = x_ref[:, :].astype(jnp.bfloat16)
            for c in range(C):
                rdma = x_rdma(c)
                rdma.start()
                x_rdmas.append(rdma)
        else:
            for c in range(C):
                out_ref[pl.ds(mine_send + c * R, R), :] = (
                    x_ref[pl.ds(my_y * H + c * R, R), :].astype(jnp.bfloat16)
                )
                rdma = x_rdma(c)
                rdma.start()
                x_rdmas.append(rdma)
            out_ref[pl.ds(mine_keep, H), :] = (
                x_ref[pl.ds((1 - my_y) * H, H), :].astype(jnp.bfloat16)
            )

        if VARIANT == "halfx":
            for c in range(C):
                x_rdmas[c].wait_recv()
            for c in range(C):
                x_rdmas[c].wait_send()
            return

        y_rdmas = []
        for c in range(C):
            x_rdmas[c].wait_recv()
            rdma = pltpu.make_async_remote_copy(
                src_ref=out_ref.at[pl.ds(x_recv + c * R, R), :],
                dst_ref=out_ref.at[pl.ds(x_recv + c * R, R), :],
                send_sem=y_send_sems.at[c],
                recv_sem=y_recv_sems.at[c],
                device_id=y_nbr,
                device_id_type=pl.DeviceIdType.MESH,
            )
            rdma.start()
            y_rdmas.append(rdma)

        for c in range(C):
            y_rdmas[c].wait_recv()
        for c in range(C):
            x_rdmas[c].wait_send()
            y_rdmas[c].wait_send()

    return pl.pallas_call(
        body,
        out_shape=jax.ShapeDtypeStruct((2 * M, N), jnp.bfloat16),
        in_specs=[pl.BlockSpec(memory_space=pltpu.VMEM)],
        out_specs=pl.BlockSpec(memory_space=pltpu.VMEM),
        scratch_shapes=[
            pltpu.SemaphoreType.DMA((C,)),
            pltpu.SemaphoreType.DMA((C,)),
            pltpu.SemaphoreType.DMA((C,)),
            pltpu.SemaphoreType.DMA((C,)),
        ],
        compiler_params=pltpu.CompilerParams(collective_id=0),
    )(x)


# device time: 58442 ns/iter; 1.1880x vs baseline; 1.0370x over previous
import os

import jax
import jax.numpy as jnp
from jax import lax
from jax.experimental import pallas as pl
from jax.experimental.pallas import tpu as pltpu

M = 4096
N = 1024
H = M // 2
C = int(os.environ.get("CHUNKS", "8"))
R = H // C
VARIANT = os.environ.get("KERNEL_VARIANT", "interleave")


QR = M // 4
C4 = 4
CR = QR // C4
HALF = N // 2


def kernel_v4(x):

    def body(x_ref, out_ref, xs_sems, xr_sems, ys_sems, yr_sems, zs_sems, zr_sems):
        my_x = lax.axis_index("x")
        my_y = lax.axis_index("y")
        my_z = lax.axis_index("z")
        x_nbr = (1 - my_x, my_y, my_z)
        y_nbr = (my_x, 1 - my_y, my_z)
        z_nbr = (my_x, my_y, 1 - my_z)

        barrier_sem = pltpu.get_barrier_semaphore()
        for nbr in (x_nbr, y_nbr, z_nbr):
            pl.semaphore_signal(
                barrier_sem, inc=1,
                device_id=nbr, device_id_type=pl.DeviceIdType.MESH,
            )
        pl.semaphore_wait(barrier_sem, 3)

        qidx = 2 * my_y + my_z
        mine = my_x * M
        theirs = (1 - my_x) * M
        inj = mine + qidx * QR
        xq = theirs + qidx * QR
        yq = theirs + (2 * (1 - my_y) + my_z) * QR
        zq = theirs + (2 * my_y + (1 - my_z)) * QR

        def rcopy(rows, cols, ssem, rsem, nbr):
            return pltpu.make_async_remote_copy(
                src_ref=out_ref.at[rows, cols],
                dst_ref=out_ref.at[rows, cols],
                send_sem=ssem,
                recv_sem=rsem,
                device_id=nbr,
                device_id_type=pl.DeviceIdType.MESH,
            )

        x_rdmas = []
        for c in range(C4):
            out_ref[pl.ds(inj + c * CR, CR), :] = (
                x_ref[pl.ds(qidx * QR + c * CR, CR), :].astype(jnp.bfloat16)
            )
            rdma = rcopy(pl.ds(inj + c * CR, CR), slice(None),
                         xs_sems.at[c], xr_sems.at[c], x_nbr)
            rdma.start()
            x_rdmas.append(rdma)

        for q in range(4):
            @pl.when(q != qidx)
            def _(q=q):
                out_ref[pl.ds(mine + q * QR, QR), :] = (
                    x_ref[q * QR:(q + 1) * QR, :].astype(jnp.bfloat16)
                )

        y_fwd, z_fwd = [], []
        for c in range(C4):
            x_rdmas[c].wait_recv()
            ry = rcopy(pl.ds(xq + c * CR, CR), slice(None),
                       ys_sems.at[c], yr_sems.at[c], y_nbr)
            ry.start()
            y_fwd.append(ry)
            rz = rcopy(pl.ds(xq + c * CR, CR), slice(None),
                       zs_sems.at[c], zr_sems.at[c], z_nbr)
            rz.start()
            z_fwd.append(rz)

        y_rel, z_rel = [], []
        for c in range(C4):
            y_fwd[c].wait_recv()
            rz = rcopy(pl.ds(yq + c * CR, CR), pl.ds(HALF, HALF),
                       zs_sems.at[C4 + c], zr_sems.at[C4 + c], z_nbr)
            rz.start()
            z_rel.append(rz)
            z_fwd[c].wait_recv()
            ry = rcopy(pl.ds(zq + c * CR, CR), pl.ds(0, HALF),
                       ys_sems.at[C4 + c], yr_sems.at[C4 + c], y_nbr)
            ry.start()
            y_rel.append(ry)

        for c in range(C4):
            y_rel[c].wait_recv()
            z_rel[c].wait_recv()
        for c in range(C4):
            x_rdmas[c].wait_send()
            y_fwd[c].wait_send()
            z_fwd[c].wait_send()
            y_rel[c].wait_send()
            z_rel[c].wait_send()

    return pl.pallas_call(
        body,
        out_shape=jax.ShapeDtypeStruct((2 * M, N), jnp.bfloat16),
        in_specs=[pl.BlockSpec(memory_space=pltpu.VMEM)],
        out_specs=pl.BlockSpec(memory_space=pltpu.VMEM),
        scratch_shapes=[
            pltpu.SemaphoreType.DMA((C4,)),
            pltpu.SemaphoreType.DMA((C4,)),
            pltpu.SemaphoreType.DMA((2 * C4,)),
            pltpu.SemaphoreType.DMA((2 * C4,)),
            pltpu.SemaphoreType.DMA((2 * C4,)),
            pltpu.SemaphoreType.DMA((2 * C4,)),
        ],
        compiler_params=pltpu.CompilerParams(collective_id=0),
    )(x)


def kernel(x):
    assert x.shape == (M, N), x.shape

    if VARIANT == "v4":
        return kernel_v4(x)

    def body(x_ref, out_ref, x_send_sems, x_recv_sems, y_send_sems, y_recv_sems):
        my_x = lax.axis_index("x")
        my_y = lax.axis_index("y")
        my_z = lax.axis_index("z")
        x_nbr = (1 - my_x, my_y, my_z)
        y_nbr = (my_x, 1 - my_y, my_z)

        barrier_sem = pltpu.get_barrier_semaphore()
        barrier_nbrs = (x_nbr,) if VARIANT == "halfx" else (x_nbr, y_nbr)
        for nbr in barrier_nbrs:
            pl.semaphore_signal(
                barrier_sem, inc=1,
                device_id=nbr, device_id_type=pl.DeviceIdType.MESH,
            )
        pl.semaphore_wait(barrier_sem, len(barrier_nbrs))

        mine_send = my_x * M + my_y * H
        mine_keep = my_x * M + (1 - my_y) * H
        x_recv = (1 - my_x) * M + my_y * H

        def x_rdma(c):
            return pltpu.make_async_remote_copy(
                src_ref=out_ref.at[pl.ds(mine_send + c * R, R), :],
                dst_ref=out_ref.at[pl.ds(mine_send + c * R, R), :],
                send_sem=x_send_sems.at[c],
                recv_sem=x_recv_sems.at[c],
                device_id=x_nbr,
                device_id_type=pl.DeviceIdType.MESH,
            )

        if VARIANT == "noop":
            out_ref[pl.ds(my_x * M, M), :] = x_ref[:, :].astype(jnp.bfloat16)
            out_ref[pl.ds((1 - my_x) * M, M), :] = x_ref[:, :].astype(jnp.bfloat16)
            return

        x_rdmas = []
        if VARIANT == "nosend":
            for c in range(C):
                rdma = x_rdma(c)
                rdma.start()
                x_rdmas.append(rdma)
            for c in range(C):
                x_rdmas[c].wait_recv()
            for c in range(C):
                x_rdmas[c].wait_send()
            return
        if VARIANT == "nosend2":
            for rep in range(2):
                rdmas = []
                for c in range(C):
                    rdma = pltpu.make_async_remote_copy(
                        src_ref=out_ref.at[pl.ds(mine_send + c * R, R), :],
                        dst_ref=out_ref.at[pl.ds(mine_send + c * R, R), :],
                        send_sem=(x_send_sems if rep == 0 else y_send_sems).at[c],
                        recv_sem=(x_recv_sems if rep == 0 else y_recv_sems).at[c],
                        device_id=x_nbr,
                        device_id_type=pl.DeviceIdType.MESH,
                    )
                    rdma.start()
                    rdmas.append(rdma)
                for c in range(C):
                    rdmas[c].wait_recv()
                for c in range(C):
                    rdmas[c].wait_send()
            return

        if VARIANT == "upfront":
            out_ref[pl.ds(my_x * M, M), :] = x_ref[:, :].astype(jnp.bfloat16)
            for c in range(C):
                rdma = x_rdma(c)
                rdma.start()
                x_rdmas.append(rdma)
        else:
            for c in range(C):
                out_ref[pl.ds(mine_send + c * R, R), :] = (
                    x_ref[pl.ds(my_y * H + c * R, R), :].astype(jnp.bfloat16)
                )
                rdma = x_rdma(c)
                rdma.start()
                x_rdmas.append(rdma)
            out_ref[pl.ds(mine_keep, H), :] = (
                x_ref[pl.ds((1 - my_y) * H, H), :].astype(jnp.bfloat16)
            )

        if VARIANT == "halfx":
            for c in range(C):
                x_rdmas[c].wait_recv()
            for c in range(C):
                x_rdmas[c].wait_send()
            return

        y_rdmas = []
        for c in range(C):
            x_rdmas[c].wait_recv()
            rdma = pltpu.make_async_remote_copy(
                src_ref=out_ref.at[pl.ds(x_recv + c * R, R), :],
                dst_ref=out_ref.at[pl.ds(x_recv + c * R, R), :],
                send_sem=y_send_sems.at[c],
                recv_sem=y_recv_sems.at[c],
                device_id=y_nbr,
                device_id_type=pl.DeviceIdType.MESH,
            )
            rdma.start()
            y_rdmas.append(rdma)

        for c in range(C):
            y_rdmas[c].wait_recv()
        for c in range(C):
            x_rdmas[c].wait_send()
            y_rdmas[c].wait_send()

    return pl.pallas_call(
        body,
        out_shape=jax.ShapeDtypeStruct((2 * M, N), jnp.bfloat16),
        in_specs=[pl.BlockSpec(memory_space=pltpu.VMEM)],
        out_specs=pl.BlockSpec(memory_space=pltpu.VMEM),
        scratch_shapes=[
            pltpu.SemaphoreType.DMA((C,)),
            pltpu.SemaphoreType.DMA((C,)),
            pltpu.SemaphoreType.DMA((C,)),
            pltpu.SemaphoreType.DMA((C,)),
        ],
        compiler_params=pltpu.CompilerParams(collective_id=0),
    )(x)


# device time: 53189 ns/iter; 1.3054x vs baseline; 1.0988x over previous
import os

import jax
import jax.numpy as jnp
from jax import lax
from jax.experimental import pallas as pl
from jax.experimental.pallas import tpu as pltpu

M = 4096
N = 1024
H = M // 2
C = int(os.environ.get("CHUNKS", "8"))
R = H // C
VARIANT = os.environ.get("KERNEL_VARIANT", "interleave")


QR = M // 4
C4 = int(os.environ.get("CHUNKS4", "4"))
CR = QR // C4
HALF = N // 2


def kernel_v4(x):

    def body(x_ref, out_ref, xs_sems, xr_sems, ys_sems, yr_sems, zs_sems, zr_sems):
        my_x = lax.axis_index("x")
        my_y = lax.axis_index("y")
        my_z = lax.axis_index("z")
        x_nbr = (1 - my_x, my_y, my_z)
        y_nbr = (my_x, 1 - my_y, my_z)
        z_nbr = (my_x, my_y, 1 - my_z)

        barrier_sem = pltpu.get_barrier_semaphore()
        for nbr in (x_nbr, y_nbr, z_nbr):
            pl.semaphore_signal(
                barrier_sem, inc=1,
                device_id=nbr, device_id_type=pl.DeviceIdType.MESH,
            )
        pl.semaphore_wait(barrier_sem, 3)

        qidx = 2 * my_y + my_z
        mine = my_x * M
        theirs = (1 - my_x) * M
        inj = mine + qidx * QR
        xq = theirs + qidx * QR
        yq = theirs + (2 * (1 - my_y) + my_z) * QR
        zq = theirs + (2 * my_y + (1 - my_z)) * QR

        def rcopy(rows, cols, ssem, rsem, nbr):
            return pltpu.make_async_remote_copy(
                src_ref=out_ref.at[rows, cols],
                dst_ref=out_ref.at[rows, cols],
                send_sem=ssem,
                recv_sem=rsem,
                device_id=nbr,
                device_id_type=pl.DeviceIdType.MESH,
            )

        x_rdmas = []
        for c in range(C4):
            out_ref[pl.ds(inj + c * CR, CR), :] = (
                x_ref[pl.ds(qidx * QR + c * CR, CR), :].astype(jnp.bfloat16)
            )
            rdma = rcopy(pl.ds(inj + c * CR, CR), slice(None),
                         xs_sems.at[c], xr_sems.at[c], x_nbr)
            rdma.start()
            x_rdmas.append(rdma)

        for q in range(4):
            @pl.when(q != qidx)
            def _(q=q):
                out_ref[pl.ds(mine + q * QR, QR), :] = (
                    x_ref[q * QR:(q + 1) * QR, :].astype(jnp.bfloat16)
                )

        y_fwd, z_fwd = [], []
        for c in range(C4):
            x_rdmas[c].wait_recv()
            ry = rcopy(pl.ds(xq + c * CR, CR), slice(None),
                       ys_sems.at[c], yr_sems.at[c], y_nbr)
            ry.start()
            y_fwd.append(ry)
            rz = rcopy(pl.ds(xq + c * CR, CR), slice(None),
                       zs_sems.at[c], zr_sems.at[c], z_nbr)
            rz.start()
            z_fwd.append(rz)

        y_rel, z_rel = [], []
        for c in range(C4):
            y_fwd[c].wait_recv()
            rz = rcopy(pl.ds(yq + c * CR, CR), pl.ds(HALF, HALF),
                       zs_sems.at[C4 + c], zr_sems.at[C4 + c], z_nbr)
            rz.start()
            z_rel.append(rz)
            z_fwd[c].wait_recv()
            ry = rcopy(pl.ds(zq + c * CR, CR), pl.ds(0, HALF),
                       ys_sems.at[C4 + c], yr_sems.at[C4 + c], y_nbr)
            ry.start()
            y_rel.append(ry)

        for c in range(C4):
            y_rel[c].wait_recv()
            z_rel[c].wait_recv()
        for c in range(C4):
            x_rdmas[c].wait_send()
            y_fwd[c].wait_send()
            z_fwd[c].wait_send()
            y_rel[c].wait_send()
            z_rel[c].wait_send()

    return pl.pallas_call(
        body,
        out_shape=jax.ShapeDtypeStruct((2 * M, N), jnp.bfloat16),
        in_specs=[pl.BlockSpec(memory_space=pltpu.VMEM)],
        out_specs=pl.BlockSpec(memory_space=pltpu.VMEM),
        scratch_shapes=[
            pltpu.SemaphoreType.DMA((C4,)),
            pltpu.SemaphoreType.DMA((C4,)),
            pltpu.SemaphoreType.DMA((2 * C4,)),
            pltpu.SemaphoreType.DMA((2 * C4,)),
            pltpu.SemaphoreType.DMA((2 * C4,)),
            pltpu.SemaphoreType.DMA((2 * C4,)),
        ],
        compiler_params=pltpu.CompilerParams(collective_id=0),
    )(x)


def kernel_v5(x):
    C = 8
    R = QR // C
    DD = 2 * R
    Y_REL = (2, 3, 4)
    Z_REL = (5, 6, 7)

    def body(x_ref, out_ref, xs_sems, xr_sems, ys_sems, yr_sems, zs_sems, zr_sems):
        my_x = lax.axis_index("x")
        my_y = lax.axis_index("y")
        my_z = lax.axis_index("z")
        x_nbr = (1 - my_x, my_y, my_z)
        y_nbr = (my_x, 1 - my_y, my_z)
        z_nbr = (my_x, my_y, 1 - my_z)

        barrier_sem = pltpu.get_barrier_semaphore()
        for nbr in (x_nbr, y_nbr, z_nbr):
            pl.semaphore_signal(
                barrier_sem, inc=1,
                device_id=nbr, device_id_type=pl.DeviceIdType.MESH,
            )
        pl.semaphore_wait(barrier_sem, 3)

        qidx = 2 * my_y + my_z
        diag = 2 * (1 - my_y) + (1 - my_z)
        mine = my_x * M
        theirs = (1 - my_x) * M
        inj = mine + qidx * QR
        xq = theirs + qidx * QR
        yq = theirs + (2 * (1 - my_y) + my_z) * QR
        zq = theirs + (2 * my_y + (1 - my_z)) * QR

        def rcopy(rows, ssem, rsem, nbr):
            return pltpu.make_async_remote_copy(
                src_ref=out_ref.at[rows, :],
                dst_ref=out_ref.at[rows, :],
                send_sem=ssem,
                recv_sem=rsem,
                device_id=nbr,
                device_id_type=pl.DeviceIdType.MESH,
            )

        x_rdmas = []
        for c in range(C):
            out_ref[pl.ds(inj + c * R, R), :] = (
                x_ref[pl.ds(qidx * QR + c * R, R), :].astype(jnp.bfloat16)
            )
            rdma = rcopy(pl.ds(inj + c * R, R), xs_sems.at[c], xr_sems.at[c], x_nbr)
            rdma.start()
            x_rdmas.append(rdma)

        for q in range(4):
            @pl.when(q != qidx)
            def _(q=q):
                out_ref[pl.ds(mine + q * QR, QR), :] = (
                    x_ref[q * QR:(q + 1) * QR, :].astype(jnp.bfloat16)
                )

        x_diag = rcopy(pl.ds(mine + diag * QR, DD), xs_sems.at[C], xr_sems.at[C], x_nbr)
        x_diag.start()

        y_fwd, z_fwd = [], []
        for c in range(C):
            x_rdmas[c].wait_recv()
            ry = rcopy(pl.ds(xq + c * R, R), ys_sems.at[c], yr_sems.at[c], y_nbr)
            ry.start()
            y_fwd.append(ry)
            rz = rcopy(pl.ds(xq + c * R, R), zs_sems.at[c], zr_sems.at[c], z_nbr)
            rz.start()
            z_fwd.append(rz)

        y_rel = {}
        for c in Y_REL:
            z_fwd[c].wait_recv()
            ry = rcopy(pl.ds(zq + c * R, R), ys_sems.at[C + c], yr_sems.at[C + c], y_nbr)
            ry.start()
            y_rel[c] = ry
        z_rel = {}
        for c in Z_REL:
            y_fwd[c].wait_recv()
            rz = rcopy(pl.ds(yq + c * R, R), zs_sems.at[C + c], zr_sems.at[C + c], z_nbr)
            rz.start()
            z_rel[c] = rz

        x_diag.wait_recv()
        for c in range(C):
            if c not in Z_REL:
                y_fwd[c].wait_recv()
            if c not in Y_REL:
                z_fwd[c].wait_recv()
        for c in Y_REL:
            y_rel[c].wait_recv()
        for c in Z_REL:
            z_rel[c].wait_recv()
        x_diag.wait_send()
        for c in range(C):
            x_rdmas[c].wait_send()
            y_fwd[c].wait_send()
            z_fwd[c].wait_send()
        for c in Y_REL:
            y_rel[c].wait_send()
        for c in Z_REL:
            z_rel[c].wait_send()

    return pl.pallas_call(
        body,
        out_shape=jax.ShapeDtypeStruct((2 * M, N), jnp.bfloat16),
        in_specs=[pl.BlockSpec(memory_space=pltpu.VMEM)],
        out_specs=pl.BlockSpec(memory_space=pltpu.VMEM),
        scratch_shapes=[
            pltpu.SemaphoreType.DMA((C + 1,)),
            pltpu.SemaphoreType.DMA((C + 1,)),
            pltpu.SemaphoreType.DMA((2 * C,)),
            pltpu.SemaphoreType.DMA((2 * C,)),
            pltpu.SemaphoreType.DMA((2 * C,)),
            pltpu.SemaphoreType.DMA((2 * C,)),
        ],
        compiler_params=pltpu.CompilerParams(collective_id=0),
    )(x)


def kernel(x):
    assert x.shape == (M, N), x.shape

    if VARIANT == "v4":
        return kernel_v4(x)
    if VARIANT == "v5":
        return kernel_v5(x)

    def body(x_ref, out_ref, x_send_sems, x_recv_sems, y_send_sems, y_recv_sems):
        my_x = lax.axis_index("x")
        my_y = lax.axis_index("y")
        my_z = lax.axis_index("z")
        x_nbr = (1 - my_x, my_y, my_z)
        y_nbr = (my_x, 1 - my_y, my_z)

        barrier_sem = pltpu.get_barrier_semaphore()
        barrier_nbrs = (x_nbr,) if VARIANT == "halfx" else (x_nbr, y_nbr)
        for nbr in barrier_nbrs:
            pl.semaphore_signal(
                barrier_sem, inc=1,
                device_id=nbr, device_id_type=pl.DeviceIdType.MESH,
            )
        pl.semaphore_wait(barrier_sem, len(barrier_nbrs))

        mine_send = my_x * M + my_y * H
        mine_keep = my_x * M + (1 - my_y) * H
        x_recv = (1 - my_x) * M + my_y * H

        def x_rdma(c):
            return pltpu.make_async_remote_copy(
                src_ref=out_ref.at[pl.ds(mine_send + c * R, R), :],
                dst_ref=out_ref.at[pl.ds(mine_send + c * R, R), :],
                send_sem=x_send_sems.at[c],
                recv_sem=x_recv_sems.at[c],
                device_id=x_nbr,
                device_id_type=pl.DeviceIdType.MESH,
            )

        if VARIANT == "noop":
            out_ref[pl.ds(my_x * M, M), :] = x_ref[:, :].astype(jnp.bfloat16)
            out_ref[pl.ds((1 - my_x) * M, M), :] = x_ref[:, :].astype(jnp.bfloat16)
            return

        x_rdmas = []
        if VARIANT == "nosend":
            for c in range(C):
                rdma = x_rdma(c)
                rdma.start()
                x_rdmas.append(rdma)
            for c in range(C):
                x_rdmas[c].wait_recv()
            for c in range(C):
                x_rdmas[c].wait_send()
            return
        if VARIANT == "nosend2":
            for rep in range(2):
                rdmas = []
                for c in range(C):
                    rdma = pltpu.make_async_remote_copy(
                        src_ref=out_ref.at[pl.ds(mine_send + c * R, R), :],
                        dst_ref=out_ref.at[pl.ds(mine_send + c * R, R), :],
                        send_sem=(x_send_sems if rep == 0 else y_send_sems).at[c],
                        recv_sem=(x_recv_sems if rep == 0 else y_recv_sems).at[c],
                        device_id=x_nbr,
                        device_id_type=pl.DeviceIdType.MESH,
                    )
                    rdma.start()
                    rdmas.append(rdma)
                for c in range(C):
                    rdmas[c].wait_recv()
                for c in range(C):
                    rdmas[c].wait_send()
            return

        if VARIANT == "upfront":
            out_ref[pl.ds(my_x * M, M), :] = x_ref[:, :].astype(jnp.bfloat16)
            for c in range(C):
                rdma = x_rdma(c)
                rdma.start()
                x_rdmas.append(rdma)
        else:
            for c in range(C):
                out_ref[pl.ds(mine_send + c * R, R), :] = (
                    x_ref[pl.ds(my_y * H + c * R, R), :].astype(jnp.bfloat16)
                )
                rdma = x_rdma(c)
                rdma.start()
                x_rdmas.append(rdma)
            out_ref[pl.ds(mine_keep, H), :] = (
                x_ref[pl.ds((1 - my_y) * H, H), :].astype(jnp.bfloat16)
            )

        if VARIANT == "halfx":
            for c in range(C):
                x_rdmas[c].wait_recv()
            for c in range(C):
                x_rdmas[c].wait_send()
            return

        y_rdmas = []
        for c in range(C):
            x_rdmas[c].wait_recv()
            rdma = pltpu.make_async_remote_copy(
                src_ref=out_ref.at[pl.ds(x_recv + c * R, R), :],
                dst_ref=out_ref.at[pl.ds(x_recv + c * R, R), :],
                send_sem=y_send_sems.at[c],
                recv_sem=y_recv_sems.at[c],
                device_id=y_nbr,
                device_id_type=pl.DeviceIdType.MESH,
            )
            rdma.start()
            y_rdmas.append(rdma)

        for c in range(C):
            y_rdmas[c].wait_recv()
        for c in range(C):
            x_rdmas[c].wait_send()
            y_rdmas[c].wait_send()

    return pl.pallas_call(
        body,
        out_shape=jax.ShapeDtypeStruct((2 * M, N), jnp.bfloat16),
        in_specs=[pl.BlockSpec(memory_space=pltpu.VMEM)],
        out_specs=pl.BlockSpec(memory_space=pltpu.VMEM),
        scratch_shapes=[
            pltpu.SemaphoreType.DMA((C,)),
            pltpu.SemaphoreType.DMA((C,)),
            pltpu.SemaphoreType.DMA((C,)),
            pltpu.SemaphoreType.DMA((C,)),
        ],
        compiler_params=pltpu.CompilerParams(collective_id=0),
    )(x)


# device time: 51902 ns/iter; 1.3377x vs baseline; 1.0248x over previous
import os

import jax
import jax.numpy as jnp
from jax import lax
from jax.experimental import pallas as pl
from jax.experimental.pallas import tpu as pltpu

M = 4096
N = 1024
H = M // 2
C = int(os.environ.get("CHUNKS", "8"))
R = H // C
VARIANT = os.environ.get("KERNEL_VARIANT", "v8")


QR = M // 4
C4 = int(os.environ.get("CHUNKS4", "4"))
CR = QR // C4
HALF = N // 2


def kernel_v4(x):

    def body(x_ref, out_ref, xs_sems, xr_sems, ys_sems, yr_sems, zs_sems, zr_sems):
        my_x = lax.axis_index("x")
        my_y = lax.axis_index("y")
        my_z = lax.axis_index("z")
        x_nbr = (1 - my_x, my_y, my_z)
        y_nbr = (my_x, 1 - my_y, my_z)
        z_nbr = (my_x, my_y, 1 - my_z)

        barrier_sem = pltpu.get_barrier_semaphore()
        for nbr in (x_nbr, y_nbr, z_nbr):
            pl.semaphore_signal(
                barrier_sem, inc=1,
                device_id=nbr, device_id_type=pl.DeviceIdType.MESH,
            )
        pl.semaphore_wait(barrier_sem, 3)

        qidx = 2 * my_y + my_z
        mine = my_x * M
        theirs = (1 - my_x) * M
        inj = mine + qidx * QR
        xq = theirs + qidx * QR
        yq = theirs + (2 * (1 - my_y) + my_z) * QR
        zq = theirs + (2 * my_y + (1 - my_z)) * QR

        def rcopy(rows, cols, ssem, rsem, nbr):
            return pltpu.make_async_remote_copy(
                src_ref=out_ref.at[rows, cols],
                dst_ref=out_ref.at[rows, cols],
                send_sem=ssem,
                recv_sem=rsem,
                device_id=nbr,
                device_id_type=pl.DeviceIdType.MESH,
            )

        x_rdmas = []
        for c in range(C4):
            out_ref[pl.ds(inj + c * CR, CR), :] = (
                x_ref[pl.ds(qidx * QR + c * CR, CR), :].astype(jnp.bfloat16)
            )
            rdma = rcopy(pl.ds(inj + c * CR, CR), slice(None),
                         xs_sems.at[c], xr_sems.at[c], x_nbr)
            rdma.start()
            x_rdmas.append(rdma)

        for q in range(4):
            @pl.when(q != qidx)
            def _(q=q):
                out_ref[pl.ds(mine + q * QR, QR), :] = (
                    x_ref[q * QR:(q + 1) * QR, :].astype(jnp.bfloat16)
                )

        y_fwd, z_fwd = [], []
        for c in range(C4):
            x_rdmas[c].wait_recv()
            ry = rcopy(pl.ds(xq + c * CR, CR), slice(None),
                       ys_sems.at[c], yr_sems.at[c], y_nbr)
            ry.start()
            y_fwd.append(ry)
            rz = rcopy(pl.ds(xq + c * CR, CR), slice(None),
                       zs_sems.at[c], zr_sems.at[c], z_nbr)
            rz.start()
            z_fwd.append(rz)

        y_rel, z_rel = [], []
        for c in range(C4):
            y_fwd[c].wait_recv()
            rz = rcopy(pl.ds(yq + c * CR, CR), pl.ds(HALF, HALF),
                       zs_sems.at[C4 + c], zr_sems.at[C4 + c], z_nbr)
            rz.start()
            z_rel.append(rz)
            z_fwd[c].wait_recv()
            ry = rcopy(pl.ds(zq + c * CR, CR), pl.ds(0, HALF),
                       ys_sems.at[C4 + c], yr_sems.at[C4 + c], y_nbr)
            ry.start()
            y_rel.append(ry)

        for c in range(C4):
            y_rel[c].wait_recv()
            z_rel[c].wait_recv()
        for c in range(C4):
            x_rdmas[c].wait_send()
            y_fwd[c].wait_send()
            z_fwd[c].wait_send()
            y_rel[c].wait_send()
            z_rel[c].wait_send()

    return pl.pallas_call(
        body,
        out_shape=jax.ShapeDtypeStruct((2 * M, N), jnp.bfloat16),
        in_specs=[pl.BlockSpec(memory_space=pltpu.VMEM)],
        out_specs=pl.BlockSpec(memory_space=pltpu.VMEM),
        scratch_shapes=[
            pltpu.SemaphoreType.DMA((C4,)),
            pltpu.SemaphoreType.DMA((C4,)),
            pltpu.SemaphoreType.DMA((2 * C4,)),
            pltpu.SemaphoreType.DMA((2 * C4,)),
            pltpu.SemaphoreType.DMA((2 * C4,)),
            pltpu.SemaphoreType.DMA((2 * C4,)),
        ],
        compiler_params=pltpu.CompilerParams(collective_id=0),
    )(x)


def kernel_v5(x):
    C = 8
    R = QR // C
    DD = 2 * R
    DOFF = 6 * R
    Y_REL = (0, 1, 2)
    Z_REL = (3, 4, 5)

    def body(x_ref, out_ref, xs_sems, xr_sems, ys_sems, yr_sems, zs_sems, zr_sems):
        my_x = lax.axis_index("x")
        my_y = lax.axis_index("y")
        my_z = lax.axis_index("z")
        x_nbr = (1 - my_x, my_y, my_z)
        y_nbr = (my_x, 1 - my_y, my_z)
        z_nbr = (my_x, my_y, 1 - my_z)

        barrier_sem = pltpu.get_barrier_semaphore()
        for nbr in (x_nbr, y_nbr, z_nbr):
            pl.semaphore_signal(
                barrier_sem, inc=1,
                device_id=nbr, device_id_type=pl.DeviceIdType.MESH,
            )
        pl.semaphore_wait(barrier_sem, 3)

        qidx = 2 * my_y + my_z
        diag = 2 * (1 - my_y) + (1 - my_z)
        mine = my_x * M
        theirs = (1 - my_x) * M
        inj = mine + qidx * QR
        xq = theirs + qidx * QR
        yq = theirs + (2 * (1 - my_y) + my_z) * QR
        zq = theirs + (2 * my_y + (1 - my_z)) * QR

        def rcopy(rows, ssem, rsem, nbr):
            return pltpu.make_async_remote_copy(
                src_ref=out_ref.at[rows, :],
                dst_ref=out_ref.at[rows, :],
                send_sem=ssem,
                recv_sem=rsem,
                device_id=nbr,
                device_id_type=pl.DeviceIdType.MESH,
            )

        x_rdmas = []
        for c in range(C):
            out_ref[pl.ds(inj + c * R, R), :] = (
                x_ref[pl.ds(qidx * QR + c * R, R), :].astype(jnp.bfloat16)
            )
            rdma = rcopy(pl.ds(inj + c * R, R), xs_sems.at[c], xr_sems.at[c], x_nbr)
            rdma.start()
            x_rdmas.append(rdma)

        for q in range(4):
            @pl.when(q != qidx)
            def _(q=q):
                out_ref[pl.ds(mine + q * QR, QR), :] = (
                    x_ref[q * QR:(q + 1) * QR, :].astype(jnp.bfloat16)
                )

        x_diag = rcopy(pl.ds(mine + diag * QR + DOFF, DD), xs_sems.at[C], xr_sems.at[C], x_nbr)
        x_diag.start()

        y_fwd, z_fwd = [], []
        for c in range(C):
            x_rdmas[c].wait_recv()
            ry = rcopy(pl.ds(xq + c * R, R), ys_sems.at[c], yr_sems.at[c], y_nbr)
            ry.start()
            y_fwd.append(ry)
            rz = rcopy(pl.ds(xq + c * R, R), zs_sems.at[c], zr_sems.at[c], z_nbr)
            rz.start()
            z_fwd.append(rz)

        y_rel = {}
        for c in Y_REL:
            z_fwd[c].wait_recv()
            ry = rcopy(pl.ds(zq + c * R, R), ys_sems.at[C + c], yr_sems.at[C + c], y_nbr)
            ry.start()
            y_rel[c] = ry
        z_rel = {}
        for c in Z_REL:
            y_fwd[c].wait_recv()
            rz = rcopy(pl.ds(yq + c * R, R), zs_sems.at[C + c], zr_sems.at[C + c], z_nbr)
            rz.start()
            z_rel[c] = rz

        x_diag.wait_recv()
        for c in range(C):
            if c not in Z_REL:
                y_fwd[c].wait_recv()
            if c not in Y_REL:
                z_fwd[c].wait_recv()
        for c in Y_REL:
            y_rel[c].wait_recv()
        for c in Z_REL:
            z_rel[c].wait_recv()
        x_diag.wait_send()
        for c in range(C):
            x_rdmas[c].wait_send()
            y_fwd[c].wait_send()
            z_fwd[c].wait_send()
        for c in Y_REL:
            y_rel[c].wait_send()
        for c in Z_REL:
            z_rel[c].wait_send()

    return pl.pallas_call(
        body,
        out_shape=jax.ShapeDtypeStruct((2 * M, N), jnp.bfloat16),
        in_specs=[pl.BlockSpec(memory_space=pltpu.VMEM)],
        out_specs=pl.BlockSpec(memory_space=pltpu.VMEM),
        scratch_shapes=[
            pltpu.SemaphoreType.DMA((C + 1,)),
            pltpu.SemaphoreType.DMA((C + 1,)),
            pltpu.SemaphoreType.DMA((2 * C,)),
            pltpu.SemaphoreType.DMA((2 * C,)),
            pltpu.SemaphoreType.DMA((2 * C,)),
            pltpu.SemaphoreType.DMA((2 * C,)),
        ],
        compiler_params=pltpu.CompilerParams(collective_id=0),
    )(x)


def kernel_v7(x):
    C = 8
    R = QR // C
    DD = 2 * R
    DOFF = 6 * R
    Y_REL = (0, 1, 2)
    Z_REL = (3, 4, 5)

    def body(x_ref, out_ref, shard, copy_sems,
             xs_sems, xr_sems, ys_sems, yr_sems, zs_sems, zr_sems):
        my_x = lax.axis_index("x")
        my_y = lax.axis_index("y")
        my_z = lax.axis_index("z")
        x_nbr = (1 - my_x, my_y, my_z)
        y_nbr = (my_x, 1 - my_y, my_z)
        z_nbr = (my_x, my_y, 1 - my_z)

        barrier_sem = pltpu.get_barrier_semaphore()
        for nbr in (x_nbr, y_nbr, z_nbr):
            pl.semaphore_signal(
                barrier_sem, inc=1,
                device_id=nbr, device_id_type=pl.DeviceIdType.MESH,
            )
        pl.semaphore_wait(barrier_sem, 3)

        qidx = 2 * my_y + my_z
        diag = 2 * (1 - my_y) + (1 - my_z)
        mine = my_x * M
        theirs = (1 - my_x) * M
        inj = mine + qidx * QR
        xq = theirs + qidx * QR
        yq = theirs + (2 * (1 - my_y) + my_z) * QR
        zq = theirs + (2 * my_y + (1 - my_z)) * QR

        def rcopy(src, dst_rows, ssem, rsem, nbr):
            return pltpu.make_async_remote_copy(
                src_ref=src,
                dst_ref=out_ref.at[dst_rows, :],
                send_sem=ssem,
                recv_sem=rsem,
                device_id=nbr,
                device_id_type=pl.DeviceIdType.MESH,
            )

        x_rdmas = []
        for c in range(C):
            shard[pl.ds(qidx * QR + c * R, R), :] = (
                x_ref[pl.ds(qidx * QR + c * R, R), :].astype(jnp.bfloat16)
            )
            rdma = rcopy(shard.at[pl.ds(qidx * QR + c * R, R), :],
                         pl.ds(inj + c * R, R),
                         xs_sems.at[c], xr_sems.at[c], x_nbr)
            rdma.start()
            x_rdmas.append(rdma)

        local_copies = []
        for q in range(4):
            @pl.when(q != qidx)
            def _(q=q):
                shard[q * QR:(q + 1) * QR, :] = (
                    x_ref[q * QR:(q + 1) * QR, :].astype(jnp.bfloat16)
                )
            cp = pltpu.make_async_copy(
                shard.at[q * QR:(q + 1) * QR, :],
                out_ref.at[pl.ds(mine + q * QR, QR), :],
                copy_sems.at[q],
            )
            cp.start()
            local_copies.append(cp)

        x_diag = rcopy(shard.at[pl.ds(diag * QR + DOFF, DD), :],
                       pl.ds(mine + diag * QR + DOFF, DD),
                       xs_sems.at[C], xr_sems.at[C], x_nbr)
        x_diag.start()

        y_fwd, z_fwd = [], []
        for c in range(C):
            x_rdmas[c].wait_recv()
            ry = rcopy(out_ref.at[pl.ds(xq + c * R, R), :], pl.ds(xq + c * R, R),
                       ys_sems.at[c], yr_sems.at[c], y_nbr)
            ry.start()
            y_fwd.append(ry)
            rz = rcopy(out_ref.at[pl.ds(xq + c * R, R), :], pl.ds(xq + c * R, R),
                       zs_sems.at[c], zr_sems.at[c], z_nbr)
            rz.start()
            z_fwd.append(rz)

        y_rel = {}
        for c in Y_REL:
            z_fwd[c].wait_recv()
            ry = rcopy(out_ref.at[pl.ds(zq + c * R, R), :], pl.ds(zq + c * R, R),
                       ys_sems.at[C + c], yr_sems.at[C + c], y_nbr)
            ry.start()
            y_rel[c] = ry
        z_rel = {}
        for c in Z_REL:
            y_fwd[c].wait_recv()
            rz = rcopy(out_ref.at[pl.ds(yq + c * R, R), :], pl.ds(yq + c * R, R),
                       zs_sems.at[C + c], zr_sems.at[C + c], z_nbr)
            rz.start()
            z_rel[c] = rz

        x_diag.wait_recv()
        for c in range(C):
            if c not in Z_REL:
                y_fwd[c].wait_recv()
            if c not in Y_REL:
                z_fwd[c].wait_recv()
        for c in Y_REL:
            y_rel[c].wait_recv()
        for c in Z_REL:
            z_rel[c].wait_recv()
        x_diag.wait_send()
        for c in range(C):
            x_rdmas[c].wait_send()
            y_fwd[c].wait_send()
            z_fwd[c].wait_send()
        for c in Y_REL:
            y_rel[c].wait_send()
        for c in Z_REL:
            z_rel[c].wait_send()
        for cp in local_copies:
            cp.wait()

    return pl.pallas_call(
        body,
        out_shape=jax.ShapeDtypeStruct((2 * M, N), jnp.bfloat16),
        in_specs=[pl.BlockSpec(memory_space=pltpu.VMEM)],
        out_specs=pl.BlockSpec(memory_space=pl.ANY),
        scratch_shapes=[
            pltpu.VMEM((M, N), jnp.bfloat16),
            pltpu.SemaphoreType.DMA((4,)),
            pltpu.SemaphoreType.DMA((C + 1,)),
            pltpu.SemaphoreType.DMA((C + 1,)),
            pltpu.SemaphoreType.DMA((2 * C,)),
            pltpu.SemaphoreType.DMA((2 * C,)),
            pltpu.SemaphoreType.DMA((2 * C,)),
            pltpu.SemaphoreType.DMA((2 * C,)),
        ],
        compiler_params=pltpu.CompilerParams(collective_id=0),
    )(x)


def kernel_v8(x):
    C = 8
    R = QR // C
    DD = 2 * R
    DOFF = 6 * R
    Y_REL = (0, 1, 2)
    Z_REL = (3, 4, 5)

    def body(x_ref, out_ref, shard, stage, in_sems, copy_sems,
             xs_sems, xr_sems, ys_sems, yr_sems, zs_sems, zr_sems):
        my_x = lax.axis_index("x")
        my_y = lax.axis_index("y")
        my_z = lax.axis_index("z")
        x_nbr = (1 - my_x, my_y, my_z)
        y_nbr = (my_x, 1 - my_y, my_z)
        z_nbr = (my_x, my_y, 1 - my_z)

        barrier_sem = pltpu.get_barrier_semaphore()
        for nbr in (x_nbr, y_nbr, z_nbr):
            pl.semaphore_signal(
                barrier_sem, inc=1,
                device_id=nbr, device_id_type=pl.DeviceIdType.MESH,
            )
        pl.semaphore_wait(barrier_sem, 3)

        qidx = 2 * my_y + my_z
        diag = 2 * (1 - my_y) + (1 - my_z)
        mine = my_x * M
        theirs = (1 - my_x) * M
        inj = mine + qidx * QR
        xq = theirs + qidx * QR
        yq = theirs + (2 * (1 - my_y) + my_z) * QR
        zq = theirs + (2 * my_y + (1 - my_z)) * QR

        def rcopy(src, dst_rows, ssem, rsem, nbr):
            return pltpu.make_async_remote_copy(
                src_ref=src,
                dst_ref=out_ref.at[dst_rows, :],
                send_sem=ssem,
                recv_sem=rsem,
                device_id=nbr,
                device_id_type=pl.DeviceIdType.MESH,
            )

        def fetch(k, q_dyn, slot):
            cp = pltpu.make_async_copy(
                x_ref.at[pl.ds(q_dyn * QR, QR), :],
                stage.at[slot],
                in_sems.at[k],
            )
            cp.start()
            return cp

        f0 = fetch(0, qidx, 0)
        f0.wait()
        f1 = fetch(1, (qidx + 1) % 4, 1)

        x_rdmas = []
        for c in range(C):
            shard[pl.ds(qidx * QR + c * R, R), :] = (
                stage[0, pl.ds(c * R, R), :].astype(jnp.bfloat16)
            )
            rdma = rcopy(shard.at[pl.ds(qidx * QR + c * R, R), :],
                         pl.ds(inj + c * R, R),
                         xs_sems.at[c], xr_sems.at[c], x_nbr)
            rdma.start()
            x_rdmas.append(rdma)
        cp_inj = pltpu.make_async_copy(
            shard.at[pl.ds(qidx * QR, QR), :],
            out_ref.at[pl.ds(inj, QR), :],
            copy_sems.at[0],
        )
        cp_inj.start()

        local_copies = [cp_inj]
        fetches = {1: f1}
        for k in (1, 2, 3):
            q_dyn = (qidx + k) % 4
            fetches[k].wait()
            if k < 3:
                fetches[k + 1] = fetch(k + 1, (qidx + k + 1) % 4, (k + 1) % 2)
            shard[pl.ds(q_dyn * QR, QR), :] = (
                stage[k % 2].astype(jnp.bfloat16)
            )
            cp = pltpu.make_async_copy(
                shard.at[pl.ds(q_dyn * QR, QR), :],
                out_ref.at[pl.ds(mine + q_dyn * QR, QR), :],
                copy_sems.at[k],
            )
            cp.start()
            local_copies.append(cp)

        x_diag = rcopy(shard.at[pl.ds(diag * QR + DOFF, DD), :],
                       pl.ds(mine + diag * QR + DOFF, DD),
                       xs_sems.at[C], xr_sems.at[C], x_nbr)
        x_diag.start()

        y_fwd, z_fwd = [], []
        for c in range(C):
            x_rdmas[c].wait_recv()
            ry = rcopy(out_ref.at[pl.ds(xq + c * R, R), :], pl.ds(xq + c * R, R),
                       ys_sems.at[c], yr_sems.at[c], y_nbr)
            ry.start()
            y_fwd.append(ry)
            rz = rcopy(out_ref.at[pl.ds(xq + c * R, R), :], pl.ds(xq + c * R, R),
                       zs_sems.at[c], zr_sems.at[c], z_nbr)
            rz.start()
            z_fwd.append(rz)

        y_rel = {}
        for c in Y_REL:
            z_fwd[c].wait_recv()
            ry = rcopy(out_ref.at[pl.ds(zq + c * R, R), :], pl.ds(zq + c * R, R),
                       ys_sems.at[C + c], yr_sems.at[C + c], y_nbr)
            ry.start()
            y_rel[c] = ry
        z_rel = {}
        for c in Z_REL:
            y_fwd[c].wait_recv()
            rz = rcopy(out_ref.at[pl.ds(yq + c * R, R), :], pl.ds(yq + c * R, R),
                       zs_sems.at[C + c], zr_sems.at[C + c], z_nbr)
            rz.start()
            z_rel[c] = rz

        x_diag.wait_recv()
        for c in range(C):
            if c not in Z_REL:
                y_fwd[c].wait_recv()
            if c not in Y_REL:
                z_fwd[c].wait_recv()
        for c in Y_REL:
            y_rel[c].wait_recv()
        for c in Z_REL:
            z_rel[c].wait_recv()
        x_diag.wait_send()
        for c in range(C):
            x_rdmas[c].wait_send()
            y_fwd[c].wait_send()
            z_fwd[c].wait_send()
        for c in Y_REL:
            y_rel[c].wait_send()
        for c in Z_REL:
            z_rel[c].wait_send()
        for cp in local_copies:
            cp.wait()

    return pl.pallas_call(
        body,
        out_shape=jax.ShapeDtypeStruct((2 * M, N), jnp.bfloat16),
        in_specs=[pl.BlockSpec(memory_space=pl.ANY)],
        out_specs=pl.BlockSpec(memory_space=pl.ANY),
        scratch_shapes=[
            pltpu.VMEM((M, N), jnp.bfloat16),
            pltpu.VMEM((2, QR, N), jnp.float32),
            pltpu.SemaphoreType.DMA((4,)),
            pltpu.SemaphoreType.DMA((4,)),
            pltpu.SemaphoreType.DMA((C + 1,)),
            pltpu.SemaphoreType.DMA((C + 1,)),
            pltpu.SemaphoreType.DMA((2 * C,)),
            pltpu.SemaphoreType.DMA((2 * C,)),
            pltpu.SemaphoreType.DMA((2 * C,)),
            pltpu.SemaphoreType.DMA((2 * C,)),
        ],
        compiler_params=pltpu.CompilerParams(collective_id=0),
    )(x)


def kernel(x):
    assert x.shape == (M, N), x.shape

    if VARIANT == "v4":
        return kernel_v4(x)
    if VARIANT == "v5":
        return kernel_v5(x)
    if VARIANT == "v7":
        return kernel_v7(x)
    if VARIANT == "v8":
        return kernel_v8(x)

    def body(x_ref, out_ref, x_send_sems, x_recv_sems, y_send_sems, y_recv_sems):
        my_x = lax.axis_index("x")
        my_y = lax.axis_index("y")
        my_z = lax.axis_index("z")
        x_nbr = (1 - my_x, my_y, my_z)
        y_nbr = (my_x, 1 - my_y, my_z)

        barrier_sem = pltpu.get_barrier_semaphore()
        barrier_nbrs = (x_nbr,) if VARIANT == "halfx" else (x_nbr, y_nbr)
        for nbr in barrier_nbrs:
            pl.semaphore_signal(
                barrier_sem, inc=1,
                device_id=nbr, device_id_type=pl.DeviceIdType.MESH,
            )
        pl.semaphore_wait(barrier_sem, len(barrier_nbrs))

        mine_send = my_x * M + my_y * H
        mine_keep = my_x * M + (1 - my_y) * H
        x_recv = (1 - my_x) * M + my_y * H

        def x_rdma(c):
            return pltpu.make_async_remote_copy(
                src_ref=out_ref.at[pl.ds(mine_send + c * R, R), :],
                dst_ref=out_ref.at[pl.ds(mine_send + c * R, R), :],
                send_sem=x_send_sems.at[c],
                recv_sem=x_recv_sems.at[c],
                device_id=x_nbr,
                device_id_type=pl.DeviceIdType.MESH,
            )

        if VARIANT == "noop":
            out_ref[pl.ds(my_x * M, M), :] = x_ref[:, :].astype(jnp.bfloat16)
            out_ref[pl.ds((1 - my_x) * M, M), :] = x_ref[:, :].astype(jnp.bfloat16)
            return

        x_rdmas = []
        if VARIANT == "nosend":
            for c in range(C):
                rdma = x_rdma(c)
                rdma.start()
                x_rdmas.append(rdma)
            for c in range(C):
                x_rdmas[c].wait_recv()
            for c in range(C):
                x_rdmas[c].wait_send()
            return
        if VARIANT == "nosend2":
            for rep in range(2):
                rdmas = []
                for c in range(C):
                    rdma = pltpu.make_async_remote_copy(
                        src_ref=out_ref.at[pl.ds(mine_send + c * R, R), :],
                        dst_ref=out_ref.at[pl.ds(mine_send + c * R, R), :],
                        send_sem=(x_send_sems if rep == 0 else y_send_sems).at[c],
                        recv_sem=(x_recv_sems if rep == 0 else y_recv_sems).at[c],
                        device_id=x_nbr,
                        device_id_type=pl.DeviceIdType.MESH,
                    )
                    rdma.start()
                    rdmas.append(rdma)
                for c in range(C):
                    rdmas[c].wait_recv()
                for c in range(C):
                    rdmas[c].wait_send()
            return

        if VARIANT == "upfront":
            out_ref[pl.ds(my_x * M, M), :] = x_ref[:, :].astype(jnp.bfloat16)
            for c in range(C):
                rdma = x_rdma(c)
                rdma.start()
                x_rdmas.append(rdma)
        else:
            for c in range(C):
                out_ref[pl.ds(mine_send + c * R, R), :] = (
                    x_ref[pl.ds(my_y * H + c * R, R), :].astype(jnp.bfloat16)
                )
                rdma = x_rdma(c)
                rdma.start()
                x_rdmas.append(rdma)
            out_ref[pl.ds(mine_keep, H), :] = (
                x_ref[pl.ds((1 - my_y) * H, H), :].astype(jnp.bfloat16)
            )

        if VARIANT == "halfx":
            for c in range(C):
                x_rdmas[c].wait_recv()
            for c in range(C):
                x_rdmas[c].wait_send()
            return

        y_rdmas = []
        for c in range(C):
            x_rdmas[c].wait_recv()
            rdma = pltpu.make_async_remote_copy(
                src_ref=out_ref.at[pl.ds(x_recv + c * R, R), :],
                dst_ref=out_ref.at[pl.ds(x_recv + c * R, R), :],
                send_sem=y_send_sems.at[c],
                recv_sem=y_recv_sems.at[c],
                device_id=y_nbr,
                device_id_type=pl.DeviceIdType.MESH,
            )
            rdma.start()
            y_rdmas.append(rdma)

        for c in range(C):
            y_rdmas[c].wait_recv()
        for c in range(C):
            x_rdmas[c].wait_send()
            y_rdmas[c].wait_send()

    return pl.pallas_call(
        body,
        out_shape=jax.ShapeDtypeStruct((2 * M, N), jnp.bfloat16),
        in_specs=[pl.BlockSpec(memory_space=pltpu.VMEM)],
        out_specs=pl.BlockSpec(memory_space=pltpu.VMEM),
        scratch_shapes=[
            pltpu.SemaphoreType.DMA((C,)),
            pltpu.SemaphoreType.DMA((C,)),
            pltpu.SemaphoreType.DMA((C,)),
            pltpu.SemaphoreType.DMA((C,)),
        ],
        compiler_params=pltpu.CompilerParams(collective_id=0),
    )(x)
